# baseline (speedup 1.0000x reference)
import functools
import jax
import jax.numpy as jnp
from jax import lax
from jax.experimental import pallas as pl
from jax.experimental.pallas import tpu as pltpu, tpu_sc as plsc

BATCH = 16384
HIST = 50
HP = 64
EMBED = 32
DICT = 1000000
LPAD = 1000064
SLAB = 512
NFULL = DICT // SLAB
TAIL = DICT - NFULL * SLAB

NC = 2
NS = 16
NW = NC * NS
LANES = 16

TPW = -(-NFULL // NW)

B_PER_W = BATCH // NW
CB = 32
NCH = B_PER_W // CB

_mesh = plsc.VectorSubcoreMesh(core_axis_name="c", subcore_axis_name="s")


@functools.partial(
    pl.kernel,
    out_type=jax.ShapeDtypeStruct((LPAD * EMBED,), jnp.float32),
    mesh=_mesh,
    compiler_params=pltpu.CompilerParams(use_tc_tiling_on_sc=True,
                                         needs_layout_passes=False,
                                         disable_bounds_checks=True),
    scratch_types=[
        pltpu.VMEM((EMBED, SLAB), jnp.float32),
        pltpu.VMEM((EMBED, SLAB), jnp.float32),
        pltpu.VMEM((SLAB * EMBED,), jnp.float32),
        pltpu.VMEM((SLAB * EMBED,), jnp.float32),
        pltpu.VMEM((TAIL * EMBED,), jnp.float32),
        pltpu.SemaphoreType.DMA,
        pltpu.SemaphoreType.DMA,
        pltpu.SemaphoreType.DMA,
    ],
)
def _relayout(tt_hbm, tail_hbm, flat_hbm, slab0, slab1, rows0, rows1, tail_v,
              sem_in, sem_out, sem_tail):
  wid = lax.axis_index("s") * NC + lax.axis_index("c")
  base = wid * TPW
  count = jnp.minimum(TPW, NFULL - base)
  lane32 = lax.iota(jnp.int32, LANES) * EMBED

  def fetch(t, slab):
    pltpu.async_copy(tt_hbm.at[:, pl.ds((base + t) * SLAB, SLAB)], slab,
                     sem_in)

  def wait_fetch(t, slab):
    pltpu.make_async_copy(tt_hbm.at[:, pl.ds(base * SLAB, SLAB)], slab,
                          sem_in).wait()

  def transpose(slab, rows):
    def body(i, _):
      off = i * LANES * EMBED
      for d in range(EMBED):
        v = slab[d, pl.ds(i * LANES, LANES)]
        plsc.store_scatter(rows, [lane32 + (off + d)], v)
      return 0
    lax.fori_loop(0, SLAB // LANES, body, 0)

  def write(t, rows):
    pltpu.async_copy(
        rows, flat_hbm.at[pl.ds((base + t) * SLAB * EMBED, SLAB * EMBED)],
        sem_out)

  def wait_write(rows):
    pltpu.make_async_copy(
        rows, flat_hbm.at[pl.ds(0, SLAB * EMBED)], sem_out).wait()

  @pl.when(count > 0)
  def _():
    fetch(0, slab0)

  def step(t, _):
    @pl.when(t + 1 < count)
    def _():
      @pl.when((t & 1) == 0)
      def _():
        fetch(t + 1, slab1)

      @pl.when((t & 1) == 1)
      def _():
        fetch(t + 1, slab0)

    @pl.when((t & 1) == 0)
    def _():
      wait_fetch(t, slab0)
      transpose(slab0, rows0)

      @pl.when(t >= 2)
      def _():
        wait_write(rows0)
      write(t, rows0)

    @pl.when((t & 1) == 1)
    def _():
      wait_fetch(t, slab1)
      transpose(slab1, rows1)

      @pl.when(t >= 2)
      def _():
        wait_write(rows1)
      write(t, rows1)
    return 0

  lax.fori_loop(0, count, step, 0)

  @pl.when(count >= 2)
  def _():
    wait_write(rows0)
    wait_write(rows1)

  @pl.when(count == 1)
  def _():
    wait_write(rows0)

  @pl.when(wid == 0)
  def _():
    pltpu.async_copy(tail_hbm, tail_v, sem_tail)
    pltpu.make_async_copy(tail_hbm, tail_v, sem_tail).wait()
    pltpu.sync_copy(tail_v,
                    flat_hbm.at[pl.ds(NFULL * SLAB * EMBED, TAIL * EMBED)])


@functools.partial(
    pl.kernel,
    out_type=jax.ShapeDtypeStruct((BATCH, EMBED), jnp.float32),
    mesh=_mesh,
    compiler_params=pltpu.CompilerParams(use_tc_tiling_on_sc=False,
                                         needs_layout_passes=False),
    scratch_types=[
        pltpu.VMEM((2, CB, 32), jnp.int32),
        pltpu.VMEM((2, CB, 32), jnp.int32),
        pltpu.VMEM((2, HP * CB), jnp.int32),
        pltpu.VMEM((2, HIST, CB, EMBED), jnp.float32),
        pltpu.VMEM((2, CB, EMBED), jnp.float32),
        pltpu.SemaphoreType.DMA,
        pltpu.SemaphoreType.DMA,
        pltpu.SemaphoreType.DMA,
        pltpu.SemaphoreType.DMA,
    ],
)
def _user_encoder(xa_hbm, xb_hbm, table_hbm, out_hbm, idx_a, idx_b, idx_t,
                  rows_v, out_v, sem_i, sem_g0, sem_g1, sem_o):
  wid = lax.axis_index("s") * NC + lax.axis_index("c")
  lane = lax.iota(jnp.int32, LANES)

  def stage(c):
    p = c & 1
    b0 = wid * B_PER_W + c * CB
    pltpu.async_copy(xa_hbm.at[pl.ds(b0, CB)], idx_a.at[p], sem_i)
    pltpu.async_copy(xb_hbm.at[pl.ds(b0, CB)], idx_b.at[p], sem_i)

  def transpose(c):
    p = c & 1
    b0 = wid * B_PER_W + c * CB
    pltpu.make_async_copy(xa_hbm.at[pl.ds(b0, CB)], idx_a.at[p],
                          sem_i).wait()
    pltpu.make_async_copy(xb_hbm.at[pl.ds(b0, CB)], idx_b.at[p],
                          sem_i).wait()
    dst = idx_t.at[p]

    def body(r, _):
      for o in (0, 16):
        va = idx_a[p, r, pl.ds(o, LANES)]
        plsc.store_scatter(dst, [(o + lane) * CB + r], va)
        vb = idx_b[p, r, pl.ds(o, LANES)]
        plsc.store_scatter(dst, [(32 + o + lane) * CB + r], vb)
      return 0
    lax.fori_loop(0, CB, body, 0)

  def fire(c, sem):
    p = c & 1

    def body(l, _):
      pltpu.async_copy(table_hbm.at[idx_t.at[p, pl.ds(l * CB, CB)]],
                       rows_v.at[p, l], sem)
      return 0
    lax.fori_loop(0, HIST, body, 0)

  def drain(c, sem):
    p = c & 1

    def body(l, _):
      pltpu.make_async_copy(table_hbm.at[idx_t.at[p, pl.ds(l * CB, CB)]],
                            rows_v.at[p, l], sem).wait()
      return 0
    lax.fori_loop(0, HIST, body, 0)

  def pool_and_write(c):
    p = c & 1

    def body(i, _):
      acc0 = rows_v[p, 0, i, 0:16]
      acc1 = rows_v[p, 0, i, 16:32]
      for l in range(1, HIST):
        acc0 = acc0 + rows_v[p, l, i, 0:16]
        acc1 = acc1 + rows_v[p, l, i, 16:32]
      scale = jnp.float32(1.0 / HIST)
      out_v[p, i, 0:16] = acc0 * scale
      out_v[p, i, 16:32] = acc1 * scale
      return 0
    lax.fori_loop(0, CB, body, 0)
    pltpu.async_copy(out_v.at[p],
                     out_hbm.at[pl.ds(wid * B_PER_W + c * CB, CB)], sem_o)

  def wait_out(c):
    pltpu.make_async_copy(out_v.at[c & 1],
                          out_hbm.at[pl.ds(wid * B_PER_W, CB)], sem_o).wait()

  stage(0)
  transpose(0)

  @pl.when(NCH > 1)
  def _():
    stage(1)

  def chunk_body(c, _):
    @pl.when(c == 0)
    def _():
      fire(0, sem_g0)

    @pl.when(c + 1 < NCH)
    def _():
      transpose(c + 1)

      @pl.when((c & 1) == 0)
      def _():
        fire(c + 1, sem_g1)

      @pl.when((c & 1) == 1)
      def _():
        fire(c + 1, sem_g0)

    @pl.when(c + 2 < NCH)
    def _():
      stage(c + 2)

    @pl.when((c & 1) == 0)
    def _():
      drain(c, sem_g0)

    @pl.when((c & 1) == 1)
    def _():
      drain(c, sem_g1)

    @pl.when(c >= 2)
    def _():
      wait_out(c)

    pool_and_write(c)
    return 0

  lax.fori_loop(0, NCH, chunk_body, 0)
  wait_out(0)
  wait_out(1)


def kernel(x, table):
  tail = table[DICT - TAIL:, :].reshape(TAIL * EMBED)
  flat = _relayout(table.T, tail)
  t2 = flat.reshape(LPAD, EMBED)
  xp = jnp.pad(x.astype(jnp.int32), ((0, 0), (0, HP - HIST)))
  return _user_encoder(xp[:, 0:32], xp[:, 32:64], t2)


# speedup vs baseline: 2.0292x; 2.0292x over previous
import functools
import jax
import jax.numpy as jnp
from jax import lax
from jax.experimental import pallas as pl
from jax.experimental.pallas import tpu as pltpu, tpu_sc as plsc

BATCH = 16384
HIST = 50
HP = 64
EMBED = 32
DICT = 1000000
LPAD = 1000064
SLAB = 512
NFULL = DICT // SLAB
TAIL = DICT - NFULL * SLAB

NC = 2
NS = 16
NW = NC * NS
LANES = 16

TPW = -(-NFULL // NW)

B_PER_W = BATCH // NW
CB = 32
NCH = B_PER_W // CB

_mesh = plsc.VectorSubcoreMesh(core_axis_name="c", subcore_axis_name="s")


@functools.partial(
    pl.kernel,
    out_type=jax.ShapeDtypeStruct((LPAD * EMBED,), jnp.float32),
    mesh=_mesh,
    compiler_params=pltpu.CompilerParams(use_tc_tiling_on_sc=True,
                                         needs_layout_passes=False,
                                         disable_bounds_checks=True),
    scratch_types=[
        pltpu.VMEM((EMBED, SLAB), jnp.float32),
        pltpu.VMEM((EMBED, SLAB), jnp.float32),
        pltpu.VMEM((SLAB * EMBED,), jnp.float32),
        pltpu.VMEM((SLAB * EMBED,), jnp.float32),
        pltpu.VMEM((TAIL * EMBED,), jnp.float32),
        pltpu.SemaphoreType.DMA,
        pltpu.SemaphoreType.DMA,
        pltpu.SemaphoreType.DMA,
    ],
)
def _relayout(tt_hbm, tail_hbm, flat_hbm, slab0, slab1, rows0, rows1, tail_v,
              sem_in, sem_out, sem_tail):
  wid = lax.axis_index("s") * NC + lax.axis_index("c")
  base = wid * TPW
  count = jnp.minimum(TPW, NFULL - base)
  lane = lax.iota(jnp.int32, LANES)

  def fetch(t, slab):
    pltpu.async_copy(tt_hbm.at[:, pl.ds((base + t) * SLAB, SLAB)], slab,
                     sem_in)

  def wait_fetch(t, slab):
    pltpu.make_async_copy(tt_hbm.at[:, pl.ds(base * SLAB, SLAB)], slab,
                          sem_in).wait()

  def transpose(slab, rows):
    def body(i, _):
      r_vec = i * LANES + lane
      base_store = r_vec * EMBED
      for d in range(EMBED):
        ddv = (lane + d) & (EMBED - 1)
        v = plsc.load_gather(slab, [ddv, r_vec])
        plsc.store_scatter(rows, [base_store + ddv], v)
      return 0
    lax.fori_loop(0, SLAB // LANES, body, 0)

  def write(t, rows):
    pltpu.async_copy(
        rows, flat_hbm.at[pl.ds((base + t) * SLAB * EMBED, SLAB * EMBED)],
        sem_out)

  def wait_write(rows):
    pltpu.make_async_copy(
        rows, flat_hbm.at[pl.ds(0, SLAB * EMBED)], sem_out).wait()

  @pl.when(count > 0)
  def _():
    fetch(0, slab0)

  def step(t, _):
    @pl.when(t + 1 < count)
    def _():
      @pl.when((t & 1) == 0)
      def _():
        fetch(t + 1, slab1)

      @pl.when((t & 1) == 1)
      def _():
        fetch(t + 1, slab0)

    @pl.when((t & 1) == 0)
    def _():
      wait_fetch(t, slab0)
      transpose(slab0, rows0)

      @pl.when(t >= 2)
      def _():
        wait_write(rows0)
      write(t, rows0)

    @pl.when((t & 1) == 1)
    def _():
      wait_fetch(t, slab1)
      transpose(slab1, rows1)

      @pl.when(t >= 2)
      def _():
        wait_write(rows1)
      write(t, rows1)
    return 0

  lax.fori_loop(0, count, step, 0)

  @pl.when(count >= 2)
  def _():
    wait_write(rows0)
    wait_write(rows1)

  @pl.when(count == 1)
  def _():
    wait_write(rows0)

  @pl.when(wid == 0)
  def _():
    pltpu.async_copy(tail_hbm, tail_v, sem_tail)
    pltpu.make_async_copy(tail_hbm, tail_v, sem_tail).wait()
    pltpu.sync_copy(tail_v,
                    flat_hbm.at[pl.ds(NFULL * SLAB * EMBED, TAIL * EMBED)])


@functools.partial(
    pl.kernel,
    out_type=jax.ShapeDtypeStruct((BATCH, EMBED), jnp.float32),
    mesh=_mesh,
    compiler_params=pltpu.CompilerParams(use_tc_tiling_on_sc=False,
                                         needs_layout_passes=False),
    scratch_types=[
        pltpu.VMEM((2, CB, 32), jnp.int32),
        pltpu.VMEM((2, CB, 32), jnp.int32),
        pltpu.VMEM((2, HP * CB), jnp.int32),
        pltpu.VMEM((2, HIST, CB, EMBED), jnp.float32),
        pltpu.VMEM((2, CB, EMBED), jnp.float32),
        pltpu.SemaphoreType.DMA,
        pltpu.SemaphoreType.DMA,
        pltpu.SemaphoreType.DMA,
        pltpu.SemaphoreType.DMA,
    ],
)
def _user_encoder(xa_hbm, xb_hbm, table_hbm, out_hbm, idx_a, idx_b, idx_t,
                  rows_v, out_v, sem_i, sem_g0, sem_g1, sem_o):
  wid = lax.axis_index("s") * NC + lax.axis_index("c")
  lane = lax.iota(jnp.int32, LANES)

  def stage(c):
    p = c & 1
    b0 = wid * B_PER_W + c * CB
    pltpu.async_copy(xa_hbm.at[pl.ds(b0, CB)], idx_a.at[p], sem_i)
    pltpu.async_copy(xb_hbm.at[pl.ds(b0, CB)], idx_b.at[p], sem_i)

  def transpose(c):
    p = c & 1
    b0 = wid * B_PER_W + c * CB
    pltpu.make_async_copy(xa_hbm.at[pl.ds(b0, CB)], idx_a.at[p],
                          sem_i).wait()
    pltpu.make_async_copy(xb_hbm.at[pl.ds(b0, CB)], idx_b.at[p],
                          sem_i).wait()
    dst = idx_t.at[p]

    def body(r, _):
      for o in (0, 16):
        va = idx_a[p, r, pl.ds(o, LANES)]
        plsc.store_scatter(dst, [(o + lane) * CB + r], va)
        vb = idx_b[p, r, pl.ds(o, LANES)]
        plsc.store_scatter(dst, [(32 + o + lane) * CB + r], vb)
      return 0
    lax.fori_loop(0, CB, body, 0)

  def fire(c, sem):
    p = c & 1

    def body(l, _):
      pltpu.async_copy(table_hbm.at[idx_t.at[p, pl.ds(l * CB, CB)]],
                       rows_v.at[p, l], sem)
      return 0
    lax.fori_loop(0, HIST, body, 0)

  def drain(c, sem):
    p = c & 1

    def body(l, _):
      pltpu.make_async_copy(table_hbm.at[idx_t.at[p, pl.ds(l * CB, CB)]],
                            rows_v.at[p, l], sem).wait()
      return 0
    lax.fori_loop(0, HIST, body, 0)

  def pool_and_write(c):
    p = c & 1

    def body(i, _):
      acc0 = rows_v[p, 0, i, 0:16]
      acc1 = rows_v[p, 0, i, 16:32]
      for l in range(1, HIST):
        acc0 = acc0 + rows_v[p, l, i, 0:16]
        acc1 = acc1 + rows_v[p, l, i, 16:32]
      scale = jnp.float32(1.0 / HIST)
      out_v[p, i, 0:16] = acc0 * scale
      out_v[p, i, 16:32] = acc1 * scale
      return 0
    lax.fori_loop(0, CB, body, 0)
    pltpu.async_copy(out_v.at[p],
                     out_hbm.at[pl.ds(wid * B_PER_W + c * CB, CB)], sem_o)

  def wait_out(c):
    pltpu.make_async_copy(out_v.at[c & 1],
                          out_hbm.at[pl.ds(wid * B_PER_W, CB)], sem_o).wait()

  stage(0)
  transpose(0)

  @pl.when(NCH > 1)
  def _():
    stage(1)

  def chunk_body(c, _):
    @pl.when(c == 0)
    def _():
      fire(0, sem_g0)

    @pl.when(c + 1 < NCH)
    def _():
      transpose(c + 1)

      @pl.when((c & 1) == 0)
      def _():
        fire(c + 1, sem_g1)

      @pl.when((c & 1) == 1)
      def _():
        fire(c + 1, sem_g0)

    @pl.when(c + 2 < NCH)
    def _():
      stage(c + 2)

    @pl.when((c & 1) == 0)
    def _():
      drain(c, sem_g0)

    @pl.when((c & 1) == 1)
    def _():
      drain(c, sem_g1)

    @pl.when(c >= 2)
    def _():
      wait_out(c)

    pool_and_write(c)
    return 0

  lax.fori_loop(0, NCH, chunk_body, 0)
  wait_out(0)
  wait_out(1)


def kernel(x, table):
  tail = table[DICT - TAIL:, :].reshape(TAIL * EMBED)
  flat = _relayout(table.T, tail)
  t2 = flat.reshape(LPAD, EMBED)
  xp = jnp.pad(x.astype(jnp.int32), ((0, 0), (0, HP - HIST)))
  return _user_encoder(xp[:, 0:32], xp[:, 32:64], t2)
